# Initial kernel scaffold; baseline (speedup 1.0000x reference)
import jax
import jax.numpy as jnp
from jax.experimental import pallas as pl


def kernel(inputs):
    raise NotImplementedError("write your pallas kernel here")


# speedup vs baseline: 8.9012x; 8.9012x over previous
import functools

import jax
import jax.numpy as jnp
from jax import lax
from jax.experimental import pallas as pl
from jax.experimental.pallas import tpu as pltpu
from jax.experimental.pallas import tpu_sc as plsc

N = 576
C = 384
BT = 32
K = 29
L = 16
NCH = C // L
UNROLL = 8

_mesh = plsc.VectorSubcoreMesh(core_axis_name="c", subcore_axis_name="s")


@functools.partial(
    pl.kernel,
    out_type=jax.ShapeDtypeStruct((BT, N, C), jnp.float32),
    mesh=_mesh,
    scratch_types=[pltpu.VMEM((N, L), jnp.float32)],
    compiler_params=pltpu.CompilerParams(use_tc_tiling_on_sc=False,
                                        needs_layout_passes=False),
)
def _wta_sc(x_hbm, out_hbm, xbuf):
    wid = lax.axis_index("s") * 2 + lax.axis_index("c")

    zeros_i = jnp.zeros((L,), jnp.int32)
    ones_f = jnp.ones((L,), jnp.float32)
    zeros_f = jnp.zeros((L,), jnp.float32)
    kvec = jnp.full((L,), K, jnp.int32)

    def chunk_body(cc, carry):
        pltpu.sync_copy(x_hbm.at[wid, :, pl.ds(cc * L, L)], xbuf)

        def bs_body(_, lohi):
            lo, hi = lohi
            mid = lax.shift_right_logical(lo + hi, 1)

            def cnt_body(ii, acc):
                base = ii * UNROLL
                for u in range(UNROLL):
                    kb = plsc.bitcast(xbuf[base + u], jnp.int32)
                    acc = acc + (kb >= mid).astype(jnp.int32)
                return acc

            cnt = lax.fori_loop(0, N // UNROLL, cnt_body, zeros_i)
            ge = cnt >= kvec
            return jnp.where(ge, mid, lo), jnp.where(ge, hi, mid)

        lo0 = zeros_i
        hi0 = jnp.full((L,), 0x7F800000, jnp.int32)
        v, _ = lax.fori_loop(0, 31, bs_body, (lo0, hi0))

        def cg_body(ii, acc):
            base = ii * UNROLL
            for u in range(UNROLL):
                kb = plsc.bitcast(xbuf[base + u], jnp.int32)
                acc = acc + (kb > v).astype(jnp.int32)
            return acc

        gcnt = lax.fori_loop(0, N // UNROLL, cg_body, zeros_i)
        need = kvec - gcnt

        def fin_body(jj, t):
            base = N - 1 - jj * UNROLL
            for u in range(UNROLL):
                i = base - u
                kb = plsc.bitcast(xbuf[i], jnp.int32)
                te = (kb == v) & (t < need)
                t = t + te.astype(jnp.int32)
                sel = ((kb > v) | te) & (kb != zeros_i)
                xbuf[i] = jnp.where(sel, ones_f, zeros_f)
            return t

        lax.fori_loop(0, N // UNROLL, fin_body, zeros_i)

        pltpu.sync_copy(xbuf, out_hbm.at[wid, :, pl.ds(cc * L, L)])
        return carry

    lax.fori_loop(0, NCH, chunk_body, 0)


def kernel(inputs):
    x = jnp.reshape(inputs, (BT, N, C))
    out = _wta_sc(x)
    return jnp.reshape(out, inputs.shape)
